# Initial kernel scaffold; baseline (speedup 1.0000x reference)
import jax
import jax.numpy as jnp
from jax.experimental import pallas as pl


def kernel(pred_flows, kps):
    raise NotImplementedError("write your pallas kernel here")


# speedup vs baseline: 3.0801x; 3.0801x over previous
import jax
import jax.numpy as jnp
from jax import lax
from jax.experimental import pallas as pl
from jax.experimental.pallas import tpu as pltpu
from jax.experimental.pallas import tpu_sc as plsc

NI = 5
BB = 4
KK = 17
HH = 384
WW = 384
LANES = 16
PTS = BB * KK
PV = 5
NCH = NI * 2
WROWS = WW // LANES
ROWS16 = NI * BB * 2 * HH * WROWS
GAMMA = 0.8


def _vsqrt(s):
    bits = plsc.bitcast(s, jnp.int32)
    magic = jnp.full((LANES,), 0x5F3759DF, jnp.int32)
    y = plsc.bitcast(magic - (bits >> 1), jnp.float32)
    for _ in range(3):
        y = y * (jnp.float32(1.5) - jnp.float32(0.5) * s * y * y)
    return s * y


def _body(pred_hbm, kps_hbm, out_hbm, kps_v, keys_v, idx_v, rows_v, res_v, sem):
    on0 = (lax.axis_index("c") == 0) & (lax.axis_index("s") == 0)

    @pl.when(on0)
    def _():
        pltpu.sync_copy(kps_hbm, kps_v)
        iota = lax.iota(jnp.int32, LANES)

        pts = []
        for v in range(PV):
            gid = iota + (v * LANES)
            inrange = gid < PTS
            g = jnp.where(inrange, gid, PTS - 1)
            b = g // KK
            k = g - b * KK
            base = b * (2 * KK * 2) + k * 2
            x0 = plsc.load_gather(kps_v, [base])
            y0 = plsc.load_gather(kps_v, [base + 1])
            x1 = plsc.load_gather(kps_v, [base + KK * 2])
            y1 = plsc.load_gather(kps_v, [base + KK * 2 + 1])
            valid = ((x0 >= 0) & (x0 < WW) & (y0 >= 0) & (y0 < HH) &
                     (x1 >= 0) & (x1 < WW) & (y1 >= 0) & (y1 < HH))
            scat = valid & inrange
            key = (b * HH + y0) * WW + x0
            keys_v[pl.ds(v * LANES, LANES)] = jnp.where(scat, key, -1)
            nz = (x1 != x0) | (y1 != y0)
            dx = (x1 - x0).astype(jnp.float32)
            dy = (y1 - y0).astype(jnp.float32)
            rb = jnp.where(scat, (b * 2 * HH + y0) * WROWS + (x0 >> 4), 0)
            lane = jnp.where(scat, x0 & (LANES - 1), 0)
            pts.append((scat & nz, dx, dy, rb, lane, jnp.where(scat, key, -1)))

        for i in range(NI):
            for c in range(2):
                off = i * (BB * 2 * HH * WROWS) + c * (HH * WROWS)
                for v in range(PV):
                    idx_v[i * 2 + c, pl.ds(v * LANES, LANES)] = pts[v][3] + off

        cps = [pltpu.async_copy(pred_hbm.at[idx_v.at[ch]], rows_v.at[ch], sem)
               for ch in range(NCH)]

        key_vecs = [p[5] for p in pts]

        keys_v[pl.ds(PV * LANES, LANES)] = jnp.full((LANES,), -1, jnp.int32)

        def dedup_step(q, dups):
            kq = jnp.broadcast_to(keys_v[pl.ds(q, LANES)][0], (LANES,))
            new = []
            for v in range(PV):
                gid = iota + (v * LANES)
                hit = (key_vecs[v] == kq) & (gid < q)
                new.append(dups[v] | hit.astype(jnp.int32))
            return tuple(new)

        zero = jnp.zeros((LANES,), jnp.int32)
        dups = lax.fori_loop(1, PTS, dedup_step, (zero,) * PV)

        for cp in cps:
            cp.wait()

        accs = [jnp.zeros((LANES,), jnp.float32) for _ in range(NI)]
        cntv = jnp.zeros((LANES,), jnp.float32)
        for v in range(PV):
            m0, dx, dy, _, lane, _ = pts[v]
            mask = m0 & (dups[v] == 0)
            cntv = cntv + jnp.where(mask, jnp.float32(1.0), jnp.float32(0.0))
            rid = iota + (v * LANES)
            for i in range(NI):
                u = plsc.load_gather(rows_v, [jnp.full((LANES,), i * 2, jnp.int32), rid, lane])
                w = plsc.load_gather(rows_v, [jnp.full((LANES,), i * 2 + 1, jnp.int32), rid, lane])
                du = u - dx
                dv = w - dy
                epe = _vsqrt(du * du + dv * dv)
                accs[i] = accs[i] + jnp.where(mask, epe, jnp.float32(0.0))

        wsum = jnp.zeros((LANES,), jnp.float32)
        for i in range(NI):
            wsum = wsum + jnp.float32(GAMMA ** (NI - 1 - i)) * accs[i]
        total = jnp.broadcast_to(jnp.sum(wsum), (LANES,))
        cnt = jnp.broadcast_to(jnp.sum(cntv), (LANES,))
        res_v[...] = total / cnt
        pltpu.sync_copy(res_v, out_hbm)


def kernel(pred_flows, kps):
    pred2d = pred_flows.reshape(ROWS16, LANES)
    kflat = kps.reshape(-1)
    mesh = plsc.VectorSubcoreMesh(core_axis_name="c", subcore_axis_name="s")
    f = pl.kernel(
        _body,
        mesh=mesh,
        compiler_params=pltpu.CompilerParams(
            needs_layout_passes=False, use_tc_tiling_on_sc=False),
        out_type=jax.ShapeDtypeStruct((LANES,), jnp.float32),
        scratch_types=[
            pltpu.VMEM((BB * 2 * KK * 2,), jnp.int32),
            pltpu.VMEM((PV * LANES + LANES,), jnp.int32),
            pltpu.VMEM((NCH, PV * LANES), jnp.int32),
            pltpu.VMEM((NCH, PV * LANES, LANES), jnp.float32),
            pltpu.VMEM((LANES,), jnp.float32),
            pltpu.SemaphoreType.DMA,
        ],
    )
    return f(pred2d, kflat)[0]


# speedup vs baseline: 3.2141x; 1.0435x over previous
import jax
import jax.numpy as jnp
from jax import lax
from jax.experimental import pallas as pl
from jax.experimental.pallas import tpu as pltpu
from jax.experimental.pallas import tpu_sc as plsc

NI = 5
BB = 4
KK = 17
HH = 384
WW = 384
LANES = 16
PTS = BB * KK
PV = 5
NCH = NI * 2
WROWS = WW // LANES
ROWS16 = NI * BB * 2 * HH * WROWS
GAMMA = 0.8


def _vsqrt(s):
    bits = plsc.bitcast(s, jnp.int32)
    magic = jnp.full((LANES,), 0x5F3759DF, jnp.int32)
    y = plsc.bitcast(magic - (bits >> 1), jnp.float32)
    for _ in range(3):
        y = y * (jnp.float32(1.5) - jnp.float32(0.5) * s * y * y)
    return s * y


def _body(pred_hbm, kps_hbm, out_hbm, kps_v, keys_v, idx_v, rows_v, res_v, sem):
    on0 = (lax.axis_index("c") == 0) & (lax.axis_index("s") == 0)

    @pl.when(on0)
    def _():
        pltpu.sync_copy(kps_hbm, kps_v)
        iota = lax.iota(jnp.int32, LANES)

        pts = []
        for v in range(PV):
            gid = iota + (v * LANES)
            inrange = gid < PTS
            g = jnp.where(inrange, gid, PTS - 1)
            b = g // KK
            k = g - b * KK
            base = b * (2 * KK * 2) + k * 2
            x0 = plsc.load_gather(kps_v, [base])
            y0 = plsc.load_gather(kps_v, [base + 1])
            x1 = plsc.load_gather(kps_v, [base + KK * 2])
            y1 = plsc.load_gather(kps_v, [base + KK * 2 + 1])
            valid = ((x0 >= 0) & (x0 < WW) & (y0 >= 0) & (y0 < HH) &
                     (x1 >= 0) & (x1 < WW) & (y1 >= 0) & (y1 < HH))
            scat = valid & inrange
            key = (b * HH + y0) * WW + x0
            keys_v[pl.ds(v * LANES, LANES)] = jnp.where(scat, key, -1)
            nz = (x1 != x0) | (y1 != y0)
            dx = (x1 - x0).astype(jnp.float32)
            dy = (y1 - y0).astype(jnp.float32)
            rb = jnp.where(scat, (b * 2 * HH + y0) * WROWS + (x0 >> 4), 0)
            lane = jnp.where(scat, x0 & (LANES - 1), 0)
            pts.append((scat & nz, dx, dy, rb, lane, jnp.where(scat, key, -1)))

        for i in range(NI):
            for c in range(2):
                off = i * (BB * 2 * HH * WROWS) + c * (HH * WROWS)
                for v in range(PV):
                    idx_v[i * 2 + c, pl.ds(v * LANES, LANES)] = pts[v][3] + off

        cps = [pltpu.async_copy(pred_hbm.at[idx_v.at[ch]], rows_v.at[ch], sem)
               for ch in range(NCH)]

        key_vecs = [p[5] for p in pts]

        keys_v[pl.ds(PV * LANES, LANES)] = jnp.full((LANES,), -1, jnp.int32)

        def dedup_step(q, dups):
            kq = jnp.broadcast_to(keys_v[pl.ds(q, LANES)][0], (LANES,))
            new = []
            for v in range(PV):
                gid = iota + (v * LANES)
                hit = (key_vecs[v] == kq) & (gid < q)
                new.append(dups[v] | hit.astype(jnp.int32))
            return tuple(new)

        zero = jnp.zeros((LANES,), jnp.int32)
        dups = lax.fori_loop(1, PTS, dedup_step, (zero,) * PV)

        for cp in cps:
            cp.wait()

        accs = [jnp.zeros((LANES,), jnp.float32) for _ in range(NI)]
        cntv = jnp.zeros((LANES,), jnp.float32)
        for v in range(PV):
            m0, dx, dy, _, lane, _ = pts[v]
            mask = m0 & (dups[v] == 0)
            cntv = cntv + jnp.where(mask, jnp.float32(1.0), jnp.float32(0.0))
            rid = iota + (v * LANES)
            for i in range(NI):
                u = plsc.load_gather(rows_v, [jnp.full((LANES,), i * 2, jnp.int32), rid, lane])
                w = plsc.load_gather(rows_v, [jnp.full((LANES,), i * 2 + 1, jnp.int32), rid, lane])
                du = u - dx
                dv = w - dy
                epe = _vsqrt(du * du + dv * dv)
                accs[i] = accs[i] + jnp.where(mask, epe, jnp.float32(0.0))

        wsum = jnp.zeros((LANES,), jnp.float32)
        for i in range(NI):
            wsum = wsum + jnp.float32(GAMMA ** (NI - 1 - i)) * accs[i]
        total = jnp.broadcast_to(jnp.sum(wsum), (LANES,))
        cnt = jnp.broadcast_to(jnp.sum(cntv), (LANES,))
        res_v[...] = total / cnt
        pltpu.sync_copy(res_v, out_hbm)


def kernel(pred_flows, kps):
    pred2d = pred_flows.reshape(ROWS16, LANES)
    kflat = kps.reshape(-1)
    mesh = plsc.VectorSubcoreMesh(
        core_axis_name="c", subcore_axis_name="s", num_cores=1)
    f = pl.kernel(
        _body,
        mesh=mesh,
        compiler_params=pltpu.CompilerParams(
            needs_layout_passes=False, use_tc_tiling_on_sc=False),
        out_type=jax.ShapeDtypeStruct((LANES,), jnp.float32),
        scratch_types=[
            pltpu.VMEM((BB * 2 * KK * 2,), jnp.int32),
            pltpu.VMEM((PV * LANES + LANES,), jnp.int32),
            pltpu.VMEM((NCH, PV * LANES), jnp.int32),
            pltpu.VMEM((NCH, PV * LANES, LANES), jnp.float32),
            pltpu.VMEM((LANES,), jnp.float32),
            pltpu.SemaphoreType.DMA,
        ],
    )
    return f(pred2d, kflat)[0]
